# Initial kernel scaffold; baseline (speedup 1.0000x reference)
import jax
import jax.numpy as jnp
from jax.experimental import pallas as pl


def kernel(features, type_mask, edge_index_00, edge_index_01, edge_index_10, edge_index_11, mp_idx_00, mp_idx_01, mp_idx_10, mp_idx_11, target_idx_0, target_idx_1, attn, fc1_w, fc1_b, fc2_w, fc2_b, fc_w, fc_b):
    raise NotImplementedError("write your pallas kernel here")


# speedup vs baseline: 35.5076x; 35.5076x over previous
import functools

import jax
import jax.numpy as jnp
from jax import lax
from jax.experimental import pallas as pl
from jax.experimental.pallas import tpu as pltpu
from jax.experimental.pallas import tpu_sc as plsc

N_TOTAL = 100000
N_G = 10000
E = 160000
B = 1024
D = 128
H = 8
HD = H * D

NC = 2
NS = 16
NW = NC * NS
EPT = 5120
EPAD = NW * EPT
NCHUNK = EPT // 512
KCAP = EPT + 16
SPW = B // NW
SCH = 512
MCAP = SCH + 16


def _iota16():
    return lax.broadcasted_iota(jnp.int32, (16,), 0)


def _p_body(x_ref, w_ref, o_ref):
    o_ref[...] = jnp.dot(x_ref[...], w_ref[...].T,
                         preferred_element_type=jnp.float32)


def _compute_p(features, attn):
    blk = 4000
    return pl.pallas_call(
        _p_body,
        grid=(N_TOTAL // blk,),
        in_specs=[pl.BlockSpec((blk, D), lambda i: (i, 0)),
                  pl.BlockSpec((H, D), lambda i: (0, 0))],
        out_specs=pl.BlockSpec((blk, H), lambda i: (i, 0)),
        out_shape=jax.ShapeDtypeStruct((N_TOTAL, H), jnp.float32),
    )(features, attn)


def _sca_body(p_hbm, dsts_hbm, mps_hbm, tgts_hbm,
              kslot_hbm, kexp_hbm, kmp_hbm, counts_hbm, denom_hbm, tslot_hbm,
              smap, tgtv, dstv, mpv, pbuf, rowidx, colbase, dacc, kslotv,
              kexpv, kmpv, cntv, tslotv, sem):
    cid = lax.axis_index("c")
    sid = lax.axis_index("s")
    wid = sid * NC + cid
    it = _iota16()

    pltpu.sync_copy(tgts_hbm, tgtv)

    for nt in range(2):
        ntv = jnp.full((16,), nt, jnp.int32)

        def initbody(i, c, nt=nt):
            smap[nt, pl.ds(i * 16, 16)] = jnp.full((16,), -1, jnp.int32)
            return c
        lax.fori_loop(0, N_G // 16, initbody, 0)

        def scatbody(b, c, nt=nt, ntv=ntv):
            bv = jnp.full((16,), b, jnp.int32)
            node = plsc.load_gather(tgtv, [bv + nt * B])
            plsc.store_scatter(smap, [ntv, node], bv, mask=it == 0)
            return c
        lax.fori_loop(0, B, scatbody, 0)

    for m in range(4):
        nt = m // 2
        ntv = jnp.full((16,), nt, jnp.int32)

        def dz(i, c):
            dacc[pl.ds(i * 16, 16)] = jnp.zeros((16,), jnp.float32)
            return c
        lax.fori_loop(0, B * H // 16, dz, 0)

        def chunk(ci, cnt, m=m, ntv=ntv):
            ebase = wid * EPT + ci * 512
            pltpu.sync_copy(dsts_hbm.at[pl.ds(m * EPAD + ebase, 512)], dstv)
            pltpu.sync_copy(
                mps_hbm.at[pl.ds((m * EPAD + ebase) * 3, 1536)], mpv)

            def group(g, cnt):
                d16 = dstv[pl.ds(g * 16, 16)]
                dcl = jnp.maximum(d16, 0)
                slot = plsc.load_gather(smap, [ntv, dcl])
                keep = (slot >= 0) & (d16 >= 0)
                rows0 = it * 3 + g * 48
                for l in range(3):
                    ml = plsc.load_gather(mpv, [rows0 + l])
                    plsc.store_compressed(
                        kmpv.at[pl.ds(cnt + l * KCAP, 16)], ml, mask=keep)
                plsc.store_compressed(kslotv.at[pl.ds(cnt, 16)], slot,
                                      mask=keep)
                return cnt + jnp.sum(keep.astype(jnp.int32))

            return lax.fori_loop(0, 32, group, cnt)

        cnt = lax.fori_loop(0, NCHUNK, chunk, jnp.int32(0))

        ng2 = (cnt + 31) // 32

        def kchunk(ci, c, cnt=cnt):
            base = ci * 32
            for l in range(3):
                for g2 in range(2):
                    v = kmpv[pl.ds(l * KCAP + base + g2 * 16, 16)]
                    v = jnp.clip(v, 0, N_TOTAL - 1)
                    rowidx[pl.ds(l * 32 + g2 * 16, 16)] = (
                        lax.shift_right_logical(v, 4))
                    colbase[pl.ds(l * 32 + g2 * 16, 16)] = (v & 15) * H
            cps = [pltpu.async_copy(p_hbm.at[rowidx.at[pl.ds(l * 32, 32)]],
                                    pbuf.at[pl.ds(l * 32, 32)], sem)
                   for l in range(3)]
            for cp in cps:
                cp.wait()
            for g2 in range(2):
                valid = (base + g2 * 16 + it) < cnt
                s16 = kslotv[pl.ds(base + g2 * 16, 16)]
                scl = jnp.clip(s16, 0, B - 1)
                cbs = [colbase[pl.ds(l * 32 + g2 * 16, 16)]
                       for l in range(3)]
                prows = [jnp.full((16,), l * 32 + g2 * 16, jnp.int32) + it
                         for l in range(3)]
                for h in range(H):
                    a = (plsc.load_gather(pbuf, [prows[0], cbs[0] + h])
                         + plsc.load_gather(pbuf, [prows[1], cbs[1] + h])
                         + plsc.load_gather(pbuf, [prows[2], cbs[2] + h])
                         ) * (1.0 / 3.0)
                    a = jnp.where(a > 0, a, 0.2 * a)
                    ev = jnp.exp(a)
                    plsc.addupdate_scatter(dacc, [scl * H + h], ev,
                                           mask=valid)
                    kexpv[pl.ds(h * KCAP + base + g2 * 16, 16)] = ev
            return c

        lax.fori_loop(0, ng2, kchunk, 0)

        pltpu.sync_copy(kslotv,
                        kslot_hbm.at[pl.ds((m * NW + wid) * KCAP, KCAP)])
        pltpu.sync_copy(
            kexpv, kexp_hbm.at[pl.ds((m * NW + wid) * H * KCAP, H * KCAP)])
        pltpu.sync_copy(
            kmpv, kmp_hbm.at[pl.ds((m * NW + wid) * 3 * KCAP, 3 * KCAP)])
        pltpu.sync_copy(
            dacc, denom_hbm.at[pl.ds((m * NW + wid) * B * H, B * H)])
        plsc.store_scatter(cntv, [it], jnp.full((16,), cnt, jnp.int32),
                           mask=it == m)

    pltpu.sync_copy(cntv, counts_hbm.at[pl.ds(wid * 16, 16)])

    @pl.when(wid == 0)
    def _():
        for nt2 in range(2):
            nt2v = jnp.full((16,), nt2, jnp.int32)

            def tg(g, c, nt2=nt2, nt2v=nt2v):
                tv = tgtv[pl.ds(nt2 * B + g * 16, 16)]
                tslotv[pl.ds(nt2 * B + g * 16, 16)] = plsc.load_gather(
                    smap, [nt2v, tv])
                return c
            lax.fori_loop(0, B // 16, tg, 0)
        pltpu.sync_copy(tslotv, tslot_hbm)


def _sc_a(P, dsts, mps, tgts):
    mesh = plsc.VectorSubcoreMesh(core_axis_name="c", subcore_axis_name="s",
                                  num_cores=NC, num_subcores=NS)
    f32, i32 = jnp.float32, jnp.int32
    fn = pl.kernel(
        _sca_body,
        out_type=[
            jax.ShapeDtypeStruct((4 * NW * KCAP,), i32),
            jax.ShapeDtypeStruct((4 * NW * H * KCAP,), f32),
            jax.ShapeDtypeStruct((4 * NW * 3 * KCAP,), i32),
            jax.ShapeDtypeStruct((NW * 16,), i32),
            jax.ShapeDtypeStruct((4 * NW * B * H,), f32),
            jax.ShapeDtypeStruct((2 * B,), i32),
        ],
        mesh=mesh,
        compiler_params=pltpu.CompilerParams(needs_layout_passes=False),
        scratch_types=[
            pltpu.VMEM((2, N_G), i32),
            pltpu.VMEM((2 * B,), i32),
            pltpu.VMEM((512,), i32),
            pltpu.VMEM((1536,), i32),
            pltpu.VMEM((96, 128), f32),
            pltpu.VMEM((96,), i32),
            pltpu.VMEM((96,), i32),
            pltpu.VMEM((B * H,), f32),
            pltpu.VMEM((KCAP,), i32),
            pltpu.VMEM((H * KCAP,), f32),
            pltpu.VMEM((3 * KCAP,), i32),
            pltpu.VMEM((16,), i32),
            pltpu.VMEM((2 * B,), i32),
            pltpu.SemaphoreType.DMA,
        ],
    )
    return fn(P, dsts, mps, tgts)


def _dm_body(d_ref, o_ref):
    o_ref[...] = 1.0 / (jnp.sum(d_ref[...], axis=1) + 1e-9)


def _denom_merge(denom):
    return pl.pallas_call(
        _dm_body,
        out_shape=jax.ShapeDtypeStruct((4, B * H), jnp.float32),
    )(denom)


def _scb_body(feat_hbm, kslot_hbm, kexp_hbm, kmp_hbm, counts_hbm, invd_hbm,
              ft_hbm,
              countsv, invdv, sslot, sexp, smp, mslot, mexp, mmp,
              rowidx, featsv, av, acc, sem):
    cid = lax.axis_index("c")
    sid = lax.axis_index("s")
    wid = sid * NC + cid
    it = _iota16()
    lo = wid * SPW

    pltpu.sync_copy(counts_hbm, countsv)

    for m in range(4):
        def za(i, c):
            plsc.store_scatter(
                acc,
                [jnp.full((16,), lax.shift_right_logical(i, 6), jnp.int32),
                 (i & 63) * 16 + it],
                jnp.zeros((16,), jnp.float32))
            return c
        lax.fori_loop(0, SPW * HD // 16, za, 0)
        pltpu.sync_copy(
            invd_hbm.at[pl.ds(m * B * H + wid * SPW * H, SPW * H)], invdv)

        def src_loop(s, c, m=m):
            cs = countsv[pl.ds(s * 16, 16)][m]
            nch = (cs + (SCH - 1)) // SCH

            def sch_loop(ci, c2, s=s, m=m, cs=cs):
                sbase = ci * SCH
                kb = m * NW * KCAP + s * KCAP + sbase
                cps = [pltpu.async_copy(
                    kslot_hbm.at[pl.ds(kb, SCH)], sslot, sem)]
                kbe = m * NW * H * KCAP + s * H * KCAP + sbase
                for h in range(H):
                    cps.append(pltpu.async_copy(
                        kexp_hbm.at[pl.ds(kbe + h * KCAP, SCH)],
                        sexp.at[pl.ds(h * SCH, SCH)], sem))
                kbm = m * NW * 3 * KCAP + s * 3 * KCAP + sbase
                for l in range(3):
                    cps.append(pltpu.async_copy(
                        kmp_hbm.at[pl.ds(kbm + l * KCAP, SCH)],
                        smp.at[pl.ds(l * SCH, SCH)], sem))
                for cp in cps:
                    cp.wait()

                def grp(g, mc):
                    valid = (sbase + g * 16 + it) < cs
                    sl = sslot[pl.ds(g * 16, 16)]
                    match = valid & (sl >= lo) & (sl < lo + SPW)
                    plsc.store_compressed(mslot.at[pl.ds(mc, 16)], sl,
                                          mask=match)
                    for l in range(3):
                        plsc.store_compressed(
                            mmp.at[pl.ds(mc + l * MCAP, 16)],
                            smp[pl.ds(l * SCH + g * 16, 16)], mask=match)
                    for h in range(H):
                        plsc.store_compressed(
                            mexp.at[pl.ds(mc + h * MCAP, 16)],
                            sexp[pl.ds(h * SCH + g * 16, 16)], mask=match)
                    return mc + jnp.sum(match.astype(jnp.int32))

                mc = lax.fori_loop(0, SCH // 16, grp, jnp.int32(0))
                nmg = (mc + 15) // 16

                def pg_loop(gi, c3, mc=mc):
                    base = gi * 16
                    vmask = (base + it) < mc
                    msl = mslot[pl.ds(base, 16)]
                    mrel = jnp.clip(msl - lo, 0, SPW - 1)
                    for l in range(3):
                        v = mmp[pl.ds(l * MCAP + base, 16)]
                        rowidx[pl.ds(l * 16, 16)] = jnp.clip(
                            v, 0, N_TOTAL - 1)
                    cps2 = [pltpu.async_copy(
                        feat_hbm.at[rowidx.at[pl.ds(l * 16, 16)]],
                        featsv.at[pl.ds(l * 16, 16)], sem)
                        for l in range(3)]
                    for h in range(H):
                        ev = mexp[pl.ds(h * MCAP + base, 16)]
                        dn = plsc.load_gather(invdv, [mrel * H + h])
                        av[pl.ds(h * 16, 16)] = jnp.where(
                            vmask, ev * dn, 0.0)
                    for cp in cps2:
                        cp.wait()

                    def ed(j, c4, base=base):
                        jv = jnp.full((16,), j, jnp.int32)
                        hv = []
                        for k in range(D // 16):
                            col = k * 16 + it
                            f0 = plsc.load_gather(featsv, [jv, col])
                            f1 = plsc.load_gather(featsv, [jv + 16, col])
                            f2 = plsc.load_gather(featsv, [jv + 32, col])
                            hv.append((f0 + f1 + f2) * (1.0 / 3.0))
                        srel = jnp.clip(
                            plsc.load_gather(mslot, [jv + base])
                            - lo, 0, SPW - 1)
                        for h in range(H):
                            ahv = plsc.load_gather(
                                av, [jnp.full((16,), h * 16, jnp.int32) + jv])
                            for k in range(D // 16):
                                plsc.addupdate_scatter(
                                    acc, [srel, jnp.full(
                                        (16,), h * D + k * 16,
                                        jnp.int32) + it],
                                    hv[k] * ahv)
                        return c4

                    lax.fori_loop(0, 16, ed, 0)
                    return c3

                lax.fori_loop(0, nmg, pg_loop, 0)
                return c2

            lax.fori_loop(0, nch, sch_loop, 0)
            return c

        lax.fori_loop(0, NW, src_loop, 0)
        pltpu.sync_copy(acc, ft_hbm.at[m, pl.ds(wid * SPW, SPW)])


def _sc_b(features, kslot, kexp, kmp, counts, invd):
    mesh = plsc.VectorSubcoreMesh(core_axis_name="c", subcore_axis_name="s",
                                  num_cores=NC, num_subcores=NS)
    f32, i32 = jnp.float32, jnp.int32
    fn = pl.kernel(
        _scb_body,
        out_type=jax.ShapeDtypeStruct((4, B, HD), f32),
        mesh=mesh,
        compiler_params=pltpu.CompilerParams(needs_layout_passes=False),
        scratch_types=[
            pltpu.VMEM((NW * 16,), i32),
            pltpu.VMEM((SPW * H,), f32),
            pltpu.VMEM((SCH,), i32),
            pltpu.VMEM((H * SCH,), f32),
            pltpu.VMEM((3 * SCH,), i32),
            pltpu.VMEM((MCAP,), i32),
            pltpu.VMEM((H * MCAP,), f32),
            pltpu.VMEM((3 * MCAP,), i32),
            pltpu.VMEM((48,), i32),
            pltpu.VMEM((48, D), f32),
            pltpu.VMEM((H * 16,), f32),
            pltpu.VMEM((SPW, HD), f32),
            pltpu.SemaphoreType.DMA,
        ],
    )
    return fn(features, kslot, kexp, kmp, counts, invd)


def _fin_body(ftp_ref, tslot_ref, fc1w_ref, fc1b_ref, fc2w_ref, fc2b_ref,
              fcw_ref, fcb_ref,
              l1_ref, l2_ref, h1_ref, h2_ref, at1_ref, at2_ref):
    iot = lax.broadcasted_iota(jnp.int32, (B, B), 1)
    outs = []
    for m in range(4):
        ft = ftp_ref[m]
        tsl = tslot_ref[m // 2]
        g = (tsl[:, None] == iot).astype(jnp.float32)
        out = jnp.dot(g, ft, preferred_element_type=jnp.float32)
        outs.append(jnp.where(out > 0, out, jnp.exp(jnp.minimum(out, 0.0))
                              - 1.0))
    betas = []
    for m in range(4):
        s = jnp.tanh(jnp.dot(outs[m], fc1w_ref[...].T,
                             preferred_element_type=jnp.float32)
                     + fc1b_ref[...])
        betas.append(jnp.sum(s * fc2w_ref[...]) * (1.0 / B)
                     + jnp.sum(fc2b_ref[...]))
    for t, (h_ref, l_ref, at_ref) in enumerate(
            [(h1_ref, l1_ref, at1_ref), (h2_ref, l2_ref, at2_ref)]):
        b0, b1 = betas[2 * t], betas[2 * t + 1]
        mx = jnp.maximum(b0, b1)
        e0 = jnp.exp(b0 - mx)
        e1 = jnp.exp(b1 - mx)
        s = e0 + e1
        w0 = e0 / s
        w1 = e1 / s
        hcomb = w0 * outs[2 * t] + w1 * outs[2 * t + 1]
        h_ref[...] = hcomb
        l_ref[...] = (jnp.dot(hcomb, fcw_ref[...].T,
                              preferred_element_type=jnp.float32)
                      + fcb_ref[...])
        at_ref[...] = jnp.where(
            lax.broadcasted_iota(jnp.int32, (1, 2), 1) == 0, w0, w1)


def _final(ftp, tslot, fc1_w, fc1_b, fc2_w, fc2_b, fc_w, fc_b):
    f32 = jnp.float32
    return pl.pallas_call(
        _fin_body,
        out_shape=[
            jax.ShapeDtypeStruct((B, D), f32),
            jax.ShapeDtypeStruct((B, D), f32),
            jax.ShapeDtypeStruct((B, HD), f32),
            jax.ShapeDtypeStruct((B, HD), f32),
            jax.ShapeDtypeStruct((1, 2), f32),
            jax.ShapeDtypeStruct((1, 2), f32),
        ],
    )(ftp, tslot, fc1_w, fc1_b, fc2_w, fc2_b, fc_w, fc_b)


def kernel(features, type_mask, edge_index_00, edge_index_01, edge_index_10,
           edge_index_11, mp_idx_00, mp_idx_01, mp_idx_10, mp_idx_11,
           target_idx_0, target_idx_1, attn, fc1_w, fc1_b, fc2_w, fc2_b,
           fc_w, fc_b):
    P = _compute_p(features, attn).reshape(N_TOTAL // 16, 16 * H)

    dst = jnp.stack([edge_index_00[1], edge_index_01[1],
                     edge_index_10[1], edge_index_11[1]])
    dsts = jnp.full((4, EPAD), -1, jnp.int32).at[:, :E].set(dst).reshape(-1)
    mp = jnp.stack([mp_idx_00.reshape(-1), mp_idx_01.reshape(-1),
                    mp_idx_10.reshape(-1), mp_idx_11.reshape(-1)])
    mps = jnp.zeros((4, EPAD * 3), jnp.int32).at[:, :E * 3].set(mp)
    mps = mps.reshape(-1)
    tgts = jnp.concatenate([target_idx_0, target_idx_1])

    kslot, kexp, kmp, counts, denom, tslot = _sc_a(P, dsts, mps, tgts)
    invd = _denom_merge(denom.reshape(4, NW, B * H)).reshape(-1)
    ftp = _sc_b(features, kslot, kexp, kmp, counts, invd)
    l1, l2, h1, h2, a1, a2 = _final(ftp, tslot.reshape(2, B), fc1_w, fc1_b,
                                    fc2_w, fc2_b, fc_w, fc_b)
    return l1, l2, h1, h2, a1.reshape(2), a2.reshape(2)
